# Initial kernel scaffold; baseline (speedup 1.0000x reference)
import jax
import jax.numpy as jnp
from jax.experimental import pallas as pl


def kernel(x, indices):
    raise NotImplementedError("write your pallas kernel here")


# speedup vs baseline: 150.5716x; 150.5716x over previous
import jax
import jax.numpy as jnp
from jax.experimental import pallas as pl


def _unpool_body(x_ref, idx_ref, s_ref, out_ref):
    x = x_ref[0]
    idx = idx_ref[0]
    s = s_ref[...]
    h, w = x.shape
    ow = 2 * w
    xs = jax.lax.dot(x, s, preferred_element_type=jnp.float32,
                     precision=jax.lax.Precision.HIGHEST)
    i = jax.lax.broadcasted_iota(jnp.int32, (h, w), 0)
    di = (idx >= (2 * i + 1) * ow).astype(jnp.float32)
    dj = (idx & 1).astype(jnp.float32)
    code = 2.0 * di + dj
    cs = jax.lax.dot(code, s, preferred_element_type=jnp.float32)
    ci = cs.astype(jnp.int32)
    qpar = jax.lax.broadcasted_iota(jnp.int32, (h, ow), 1) & 1
    zero = jnp.zeros_like(xs)
    even = jnp.where(ci == qpar, xs, zero)
    odd = jnp.where(ci == qpar + 2, xs, zero)
    out_ref[0] = jnp.stack([even, odd], axis=1).reshape(2 * h, ow)


def kernel(x, indices):
    n, c, h, w = x.shape
    oh, ow = 2 * h, 2 * w
    nc = n * c
    xf = x.reshape(nc, h, w)
    idxf = indices.reshape(nc, h, w)
    s = (jnp.arange(ow, dtype=jnp.int32)[None, :] // 2
         == jnp.arange(w, dtype=jnp.int32)[:, None]).astype(jnp.float32)
    out = pl.pallas_call(
        _unpool_body,
        grid=(nc,),
        in_specs=[
            pl.BlockSpec((1, h, w), lambda p: (p, 0, 0)),
            pl.BlockSpec((1, h, w), lambda p: (p, 0, 0)),
            pl.BlockSpec((w, ow), lambda p: (0, 0)),
        ],
        out_specs=pl.BlockSpec((1, oh, ow), lambda p: (p, 0, 0)),
        out_shape=jax.ShapeDtypeStruct((nc, oh, ow), x.dtype),
    )(xf, idxf, s)
    return out.reshape(n, c, oh, ow)
